# Initial kernel scaffold; baseline (speedup 1.0000x reference)
import jax
import jax.numpy as jnp
from jax.experimental import pallas as pl


def kernel(s_feats, t_feats, labels, teacher_predict, student_predict, queue, epoch, max_region_num):
    raise NotImplementedError("write your pallas kernel here")


# speedup vs baseline: 41.0425x; 41.0425x over previous
import functools

import jax
import jax.numpy as jnp
from jax import lax
from jax.experimental import pallas as pl
from jax.experimental.pallas import tpu as pltpu
from jax.experimental.pallas import tpu_sc as plsc

WIN = 8
K_PER = 16
NUM_CLASSES = 3
QUEUE_SIZE = 4096
CONTRAST_SIZE = 512
DIM = 32 * WIN * WIN
T_KD = 1.0
T_C = 0.1
WARMUP = 10

B = 4
H = 384
W = 384
NMAP = B * NUM_CLASSES
NREG = NMAP * K_PER
NCON = NUM_CLASSES * CONTRAST_SIZE
PLANE = H * W

SC_CORES = 2
SC_SUBCORES = 16
SC_WORKERS = SC_CORES * SC_SUBCORES
REG_PER_WORKER = NREG // SC_WORKERS
IDX_CHUNK = 128
N_CHUNK = DIM // IDX_CHUNK


def _screen_body(lesion_ref, vs_ref, fidx_ref, pidx_ref):
    nz = (lesion_ref[...] > 0.0).astype(jnp.float32)

    acc = nz
    for s in range(1, WIN):
        acc = acc + jnp.roll(nz, -s, axis=2)
    cnt = acc
    for s in range(1, WIN):
        cnt = cnt + jnp.roll(acc, -s, axis=1)
    ratio = cnt * (1.0 / (WIN * WIN))

    row_i = lax.broadcasted_iota(jnp.int32, (1, H, 1), 1)
    col_i = lax.broadcasted_iota(jnp.int32, (1, 1, W), 2)
    Hs = H - WIN + 1
    valid = (row_i < Hs) & (col_i < Hs)
    m_i = lax.broadcasted_iota(jnp.int32, (NMAP, 1, 1), 0)
    thr = jnp.where(m_i % NUM_CLASSES == 2, jnp.float32(0.6), jnp.float32(0.8))
    neg_inf = jnp.float32(-jnp.inf)
    scores0 = jnp.where(valid, jnp.where(ratio > thr, ratio, -1.0), neg_inf)

    lane16 = lax.broadcasted_iota(jnp.int32, (NMAP, K_PER), 1)

    def body(it, carry):
        scores, ys, xs, vsv = carry
        rowmax = jnp.max(scores, axis=2, keepdims=True)
        v = jnp.max(rowmax, axis=1, keepdims=True)
        ri = jnp.min(jnp.where(rowmax == v, row_i, H), axis=1, keepdims=True)
        colhit = (scores == v) & (row_i == ri)
        colcand = jnp.where(colhit, col_i, W)
        ci = jnp.min(jnp.min(colcand, axis=2, keepdims=True), axis=1,
                     keepdims=True)
        ov = (jnp.abs(row_i - ri) < WIN) & (jnp.abs(col_i - ci) < WIN)
        scores = jnp.where(ov, neg_inf, scores)
        sel = lane16 == it
        ys = jnp.where(sel, ri[:, :, 0], ys)
        xs = jnp.where(sel, ci[:, :, 0], xs)
        vsv = jnp.where(sel, v[:, :, 0], vsv)
        return scores, ys, xs, vsv

    zi = jnp.zeros((NMAP, K_PER), jnp.int32)
    zf = jnp.zeros((NMAP, K_PER), jnp.float32)
    _, ys, xs, vsv = lax.fori_loop(0, K_PER, body, (scores0, zi, zi, zf))

    oh_m = (lax.broadcasted_iota(jnp.int32, (NREG, NMAP), 1)
            == lax.broadcasted_iota(jnp.int32, (NREG, NMAP), 0) // K_PER
            ).astype(jnp.float32)
    oh_k = (lax.broadcasted_iota(jnp.int32, (NREG, K_PER), 1)
            == lax.broadcasted_iota(jnp.int32, (NREG, K_PER), 0) % K_PER
            ).astype(jnp.float32)
    dn2 = (((1,), (0,)), ((), ()))

    def flatten_sel(a):
        rows = lax.dot_general(oh_m, a, dn2,
                               preferred_element_type=jnp.float32)
        return jnp.sum(rows * oh_k, axis=1, keepdims=True)

    vr = flatten_sel(vsv)
    vs_ref[...] = vr
    keepr = vr > 0
    yr = jnp.where(keepr, flatten_sel(ys.astype(jnp.float32)), 0.0
                   ).astype(jnp.int32)
    xr = jnp.where(keepr, flatten_sel(xs.astype(jnp.float32)), 0.0
                   ).astype(jnp.int32)

    i2 = lax.broadcasted_iota(jnp.int32, (NREG, DIM), 0)
    e2 = lax.broadcasted_iota(jnp.int32, (NREG, DIM), 1)
    b2 = i2 // (NUM_CLASSES * K_PER)
    c2 = e2 // (WIN * WIN)
    r2 = (e2 // WIN) % WIN
    k2 = e2 % WIN
    fidx_ref[...] = ((b2 * 32 + c2) * H + (yr + r2)) * W + (xr + k2)

    i3 = lax.broadcasted_iota(jnp.int32, (NREG, WIN * WIN), 0)
    e3 = lax.broadcasted_iota(jnp.int32, (NREG, WIN * WIN), 1)
    b3 = i3 // (NUM_CLASSES * K_PER)
    l3 = (i3 // K_PER) % NUM_CLASSES
    r3 = e3 // WIN
    k3 = e3 % WIN
    pidx_ref[...] = (b3 * 5 + l3 + 1) * PLANE + (yr + r3) * W + (xr + k3)


def _screen_call(lesion, interpret=False):
    return pl.pallas_call(
        _screen_body,
        out_shape=[
            jax.ShapeDtypeStruct((NREG, 1), jnp.float32),
            jax.ShapeDtypeStruct((NREG, DIM), jnp.int32),
            jax.ShapeDtypeStruct((NREG, WIN * WIN), jnp.int32),
        ],
        interpret=interpret,
    )(lesion)


def _sc_gather_body(fidx_hbm, pidx_hbm, s_hbm, t_hbm, pr_hbm, lb_hbm,
                    so_hbm, to_hbm, spo_hbm, gto_hbm,
                    idxv, pidxv, sbuf, tbuf, pbuf, gbuf, sem):
    wid = lax.axis_index("s") * SC_CORES + lax.axis_index("c")

    def do_region(j, _):
        i = wid * REG_PER_WORKER + j
        pltpu.sync_copy(fidx_hbm.at[i], idxv)
        pltpu.sync_copy(pidx_hbm.at[i], pidxv)
        cps = []
        for cc in range(N_CHUNK):
            cps.append(pltpu.async_copy(s_hbm.at[idxv.at[cc]], sbuf.at[cc], sem))
            cps.append(pltpu.async_copy(t_hbm.at[idxv.at[cc]], tbuf.at[cc], sem))
        cps.append(pltpu.async_copy(pr_hbm.at[pidxv], pbuf, sem))
        cps.append(pltpu.async_copy(lb_hbm.at[pidxv], gbuf, sem))
        for cp in cps:
            cp.wait()
        pltpu.sync_copy(sbuf, so_hbm.at[i])
        pltpu.sync_copy(tbuf, to_hbm.at[i])
        pltpu.sync_copy(pbuf, spo_hbm.at[i])
        pltpu.sync_copy(gbuf, gto_hbm.at[i])
        return 0

    lax.fori_loop(0, REG_PER_WORKER, do_region, 0)


def _sc_gather(fidx3, pidx, s_flat, t_flat, pr_flat, lb_flat):
    mesh = plsc.VectorSubcoreMesh(core_axis_name="c", subcore_axis_name="s")
    f = pl.kernel(
        _sc_gather_body,
        out_type=[
            jax.ShapeDtypeStruct((NREG, N_CHUNK, IDX_CHUNK), jnp.float32),
            jax.ShapeDtypeStruct((NREG, N_CHUNK, IDX_CHUNK), jnp.float32),
            jax.ShapeDtypeStruct((NREG, WIN * WIN), jnp.float32),
            jax.ShapeDtypeStruct((NREG, WIN * WIN), jnp.float32),
        ],
        mesh=mesh,
        scratch_types=[
            pltpu.VMEM((N_CHUNK, IDX_CHUNK), jnp.int32),
            pltpu.VMEM((WIN * WIN,), jnp.int32),
            pltpu.VMEM((N_CHUNK, IDX_CHUNK), jnp.float32),
            pltpu.VMEM((N_CHUNK, IDX_CHUNK), jnp.float32),
            pltpu.VMEM((WIN * WIN,), jnp.float32),
            pltpu.VMEM((WIN * WIN,), jnp.float32),
            pltpu.SemaphoreType.DMA,
        ],
    )
    return f(fidx3, pidx, s_flat, t_flat, pr_flat, lb_flat)


def _finish_body(sp_ref, tp_ref, spd_ref, gt_ref, vs_ref, xq_ref, out_ref):
    npix = WIN * WIN

    def region_vecs(ref):
        v = ref[...]
        v2 = v * v
        ss = v2[:, :npix]
        for c in range(1, 32):
            ss = ss + v2[:, c * npix:(c + 1) * npix]
        div = jnp.tile(jnp.sqrt(ss) + 1e-12, (1, 32))
        v = v / div
        rn = jnp.sqrt(jnp.sum(v * v, axis=1, keepdims=True))
        return v / (rn + 1e-12)

    sreg = region_vecs(sp_ref)
    treg = region_vecs(tp_ref)

    X = xq_ref[...]
    dn = (((1,), (1,)), ((), ()))
    t_c = jnp.float32(T_C)
    slog = lax.dot_general(sreg, X, dn,
                           preferred_element_type=jnp.float32) / t_c
    tlog = lax.dot_general(treg, X, dn,
                           preferred_element_type=jnp.float32) / t_c

    def logsoftmax(z):
        z = z - jnp.max(z, axis=1, keepdims=True)
        return z - jnp.log(jnp.sum(jnp.exp(z), axis=1, keepdims=True))

    log_ps = logsoftmax(slog)
    log_pt = logsoftmax(tlog)
    pt = jnp.exp(log_pt)
    kl = jnp.sum(pt * (log_pt - log_ps), axis=1, keepdims=True)

    p = jnp.clip(spd_ref[...], 1e-6, 1.0 - 1e-6)
    g = gt_ref[...]
    bce = -jnp.mean(g * jnp.log(p) + (1.0 - g) * jnp.log(1.0 - p),
                    axis=1, keepdims=True)
    inter = jnp.sum(p * g, axis=1, keepdims=True)
    dice = 1.0 - (2.0 * inter + 1e-5) / (
        jnp.sum(p, axis=1, keepdims=True)
        + jnp.sum(g, axis=1, keepdims=True) + 1e-5)
    st = 0.5 * bce + 0.5 * dice

    keep = vs_ref[...] > 0
    inf = jnp.float32(jnp.inf)
    xv = jnp.where(keep, st, inf)
    n_keep = jnp.sum(keep.astype(jnp.int32))
    q = jnp.clip((n_keep.astype(jnp.float32) * 0.6).astype(jnp.int32),
                 0, NREG - 1)
    eye = (lax.broadcasted_iota(jnp.int32, (NREG, NREG), 0)
           == lax.broadcasted_iota(jnp.int32, (NREG, NREG), 1))
    xt = jnp.sum(jnp.where(eye, xv, 0.0), axis=0, keepdims=True)
    cl = jnp.sum((xt < xv).astype(jnp.int32), axis=1, keepdims=True)
    ce = jnp.sum((xt == xv).astype(jnp.int32), axis=1, keepdims=True)
    okq = (cl <= q) & (q < cl + ce)
    min_c = jnp.min(jnp.where(okq, xv, inf))
    max_c = jnp.max(jnp.where(keep, st, -inf))
    msk = (keep & (st >= min_c) & (st <= max_c)).astype(jnp.float32)
    loss = jnp.sum(kl * msk) / jnp.maximum(jnp.sum(msk), 1.0)
    out_ref[...] = jnp.reshape(loss * jnp.float32(T_KD * T_KD), (1, 1))


def _finish_call(s_p, t_p, sp_p, gt_p, vs, xq, interpret=False):
    return pl.pallas_call(
        _finish_body,
        out_shape=jax.ShapeDtypeStruct((1, 1), jnp.float32),
        interpret=interpret,
    )(s_p, t_p, sp_p, gt_p, vs, xq)


def kernel(s_feats, t_feats, labels, teacher_predict, student_predict,
           queue, epoch, max_region_num):
    lesion = labels[:, 1:1 + NUM_CLASSES].reshape(NMAP, H, W)
    vs, fidx, pidx = _screen_call(lesion)

    fidx3 = fidx.reshape(NREG, N_CHUNK, IDX_CHUNK)
    s_p, t_p, sp_p, gt_p = _sc_gather(
        fidx3, pidx,
        s_feats.reshape(-1), t_feats.reshape(-1),
        student_predict.reshape(-1), labels.reshape(-1))

    xq = queue[:, :CONTRAST_SIZE, :].reshape(NCON, DIM)
    res = _finish_call(s_p.reshape(NREG, DIM), t_p.reshape(NREG, DIM),
                       sp_p, gt_p, vs, xq)
    loss = res[0, 0]
    return jnp.where(epoch < WARMUP, jnp.float32(1e-9), loss)


# speedup vs baseline: 47.5427x; 1.1584x over previous
import functools

import jax
import jax.numpy as jnp
from jax import lax
from jax.experimental import pallas as pl
from jax.experimental.pallas import tpu as pltpu
from jax.experimental.pallas import tpu_sc as plsc

WIN = 8
K_PER = 16
NUM_CLASSES = 3
QUEUE_SIZE = 4096
CONTRAST_SIZE = 512
DIM = 32 * WIN * WIN
T_KD = 1.0
T_C = 0.1
WARMUP = 10

B = 4
H = 384
W = 384
NMAP = B * NUM_CLASSES
NREG = NMAP * K_PER
NCON = NUM_CLASSES * CONTRAST_SIZE
PLANE = H * W

SC_CORES = 2
SC_SUBCORES = 16
SC_WORKERS = SC_CORES * SC_SUBCORES
REG_PER_WORKER = NREG // SC_WORKERS
IDX_CHUNK = 128
N_CHUNK = DIM // IDX_CHUNK


def _screen_body(labels_ref, vs_ref, fidx_ref, pidx_ref):
    lesion = labels_ref[:, 1:1 + NUM_CLASSES].reshape(NMAP, H, W)
    nz = (lesion > 0.0).astype(jnp.float32)

    a = nz + jnp.roll(nz, -1, axis=2)
    a = a + jnp.roll(a, -2, axis=2)
    a = a + jnp.roll(a, -4, axis=2)
    cnt = a + jnp.roll(a, -1, axis=1)
    cnt = cnt + jnp.roll(cnt, -2, axis=1)
    cnt = cnt + jnp.roll(cnt, -4, axis=1)

    row_i = lax.broadcasted_iota(jnp.int32, (1, H, 1), 1)
    col_i = lax.broadcasted_iota(jnp.int32, (1, 1, W), 2)
    Hs = H - WIN + 1
    valid = (row_i < Hs) & (col_i < Hs)
    m_i = lax.broadcasted_iota(jnp.int32, (NMAP, 1, 1), 0)
    thr64 = jnp.where(m_i % NUM_CLASSES == 2,
                      jnp.float32(0.6) * jnp.float32(64.0),
                      jnp.float32(0.8) * jnp.float32(64.0))
    neg_inf = jnp.float32(-jnp.inf)

    C = jnp.float32(262144.0)
    flat_f = (row_i * W + col_i).astype(jnp.float32)
    kval = jnp.where(cnt > thr64, cnt, -1.0)
    enc = jnp.where(valid, kval * C - flat_f, neg_inf)

    lane16 = lax.broadcasted_iota(jnp.int32, (NMAP, K_PER), 1)
    row_f = row_i.astype(jnp.float32)
    col_f = col_i.astype(jnp.float32)

    def body(it, carry):
        enc, colmax, ys, xs, vsv = carry
        m = jnp.max(colmax, axis=2, keepdims=True)
        k = jnp.ceil(m * (1.0 / 262144.0))
        flat = k * C - m
        y = jnp.floor(flat / jnp.float32(W))
        x = flat - y * W
        v = jnp.where(k >= 0, k * (1.0 / (WIN * WIN)), -1.0)
        ov = (jnp.abs(row_f - y) < WIN) & (jnp.abs(col_f - x) < WIN)
        enc = jnp.where(ov, neg_inf, enc)
        colmax = jnp.max(enc, axis=1, keepdims=True)
        sel = lane16 == it
        ys = jnp.where(sel, y[:, :, 0], ys)
        xs = jnp.where(sel, x[:, :, 0], xs)
        vsv = jnp.where(sel, v[:, :, 0], vsv)
        return enc, colmax, ys, xs, vsv

    zf = jnp.zeros((NMAP, K_PER), jnp.float32)
    colmax0 = jnp.max(enc, axis=1, keepdims=True)
    _, _, ys, xs, vsv = lax.fori_loop(0, K_PER, body,
                                      (enc, colmax0, zf, zf, zf))

    oh_m = (lax.broadcasted_iota(jnp.int32, (NREG, NMAP), 1)
            == lax.broadcasted_iota(jnp.int32, (NREG, NMAP), 0) // K_PER
            ).astype(jnp.float32)
    oh_k = (lax.broadcasted_iota(jnp.int32, (NREG, K_PER), 1)
            == lax.broadcasted_iota(jnp.int32, (NREG, K_PER), 0) % K_PER
            ).astype(jnp.float32)
    dn2 = (((1,), (0,)), ((), ()))

    def flatten_sel(a):
        rows = lax.dot_general(oh_m, a, dn2,
                               preferred_element_type=jnp.float32)
        return jnp.sum(rows * oh_k, axis=1, keepdims=True)

    vr = flatten_sel(vsv)
    vs_ref[...] = vr
    keepr = vr > 0
    yr = jnp.where(keepr, flatten_sel(ys), 0.0).astype(jnp.int32)
    xr = jnp.where(keepr, flatten_sel(xs), 0.0).astype(jnp.int32)

    i2 = lax.broadcasted_iota(jnp.int32, (NREG, DIM), 0)
    e2 = lax.broadcasted_iota(jnp.int32, (NREG, DIM), 1)
    b2 = i2 // (NUM_CLASSES * K_PER)
    c2 = e2 // (WIN * WIN)
    r2 = (e2 // WIN) % WIN
    k2 = e2 % WIN
    fidx_ref[...] = ((b2 * 32 + c2) * H + (yr + r2)) * W + (xr + k2)

    i3 = lax.broadcasted_iota(jnp.int32, (NREG, WIN * WIN), 0)
    e3 = lax.broadcasted_iota(jnp.int32, (NREG, WIN * WIN), 1)
    b3 = i3 // (NUM_CLASSES * K_PER)
    l3 = (i3 // K_PER) % NUM_CLASSES
    r3 = e3 // WIN
    k3 = e3 % WIN
    pidx_ref[...] = (b3 * 5 + l3 + 1) * PLANE + (yr + r3) * W + (xr + k3)


def _screen_call(labels, interpret=False):
    return pl.pallas_call(
        _screen_body,
        out_shape=[
            jax.ShapeDtypeStruct((NREG, 1), jnp.float32),
            jax.ShapeDtypeStruct((NREG, DIM), jnp.int32),
            jax.ShapeDtypeStruct((NREG, WIN * WIN), jnp.int32),
        ],
        interpret=interpret,
    )(labels)


def _sc_gather_body(fidx_hbm, pidx_hbm, s_hbm, t_hbm, pr_hbm, lb_hbm,
                    so_hbm, to_hbm, spo_hbm, gto_hbm,
                    idxv, pidxv, sbuf, tbuf, pbuf, gbuf, sem):
    wid = lax.axis_index("s") * SC_CORES + lax.axis_index("c")

    def do_region(j, _):
        i = wid * REG_PER_WORKER + j
        pltpu.sync_copy(fidx_hbm.at[i], idxv)
        pltpu.sync_copy(pidx_hbm.at[i], pidxv)
        cps = []
        for cc in range(N_CHUNK):
            cps.append(pltpu.async_copy(s_hbm.at[idxv.at[cc]], sbuf.at[cc], sem))
            cps.append(pltpu.async_copy(t_hbm.at[idxv.at[cc]], tbuf.at[cc], sem))
        cps.append(pltpu.async_copy(pr_hbm.at[pidxv], pbuf, sem))
        cps.append(pltpu.async_copy(lb_hbm.at[pidxv], gbuf, sem))
        for cp in cps:
            cp.wait()
        pltpu.sync_copy(sbuf, so_hbm.at[i])
        pltpu.sync_copy(tbuf, to_hbm.at[i])
        pltpu.sync_copy(pbuf, spo_hbm.at[i])
        pltpu.sync_copy(gbuf, gto_hbm.at[i])
        return 0

    lax.fori_loop(0, REG_PER_WORKER, do_region, 0)


def _sc_gather(fidx3, pidx, s_flat, t_flat, pr_flat, lb_flat):
    mesh = plsc.VectorSubcoreMesh(core_axis_name="c", subcore_axis_name="s")
    f = pl.kernel(
        _sc_gather_body,
        out_type=[
            jax.ShapeDtypeStruct((NREG, N_CHUNK, IDX_CHUNK), jnp.float32),
            jax.ShapeDtypeStruct((NREG, N_CHUNK, IDX_CHUNK), jnp.float32),
            jax.ShapeDtypeStruct((NREG, WIN * WIN), jnp.float32),
            jax.ShapeDtypeStruct((NREG, WIN * WIN), jnp.float32),
        ],
        mesh=mesh,
        scratch_types=[
            pltpu.VMEM((N_CHUNK, IDX_CHUNK), jnp.int32),
            pltpu.VMEM((WIN * WIN,), jnp.int32),
            pltpu.VMEM((N_CHUNK, IDX_CHUNK), jnp.float32),
            pltpu.VMEM((N_CHUNK, IDX_CHUNK), jnp.float32),
            pltpu.VMEM((WIN * WIN,), jnp.float32),
            pltpu.VMEM((WIN * WIN,), jnp.float32),
            pltpu.SemaphoreType.DMA,
        ],
    )
    return f(fidx3, pidx, s_flat, t_flat, pr_flat, lb_flat)


def _finish_body(sp_ref, tp_ref, spd_ref, gt_ref, vs_ref, xq_ref, out_ref):
    npix = WIN * WIN

    def region_vecs(ref):
        v = ref[...]
        v2 = v * v
        ss = v2[:, :npix]
        for c in range(1, 32):
            ss = ss + v2[:, c * npix:(c + 1) * npix]
        div = jnp.tile(jnp.sqrt(ss) + 1e-12, (1, 32))
        v = v / div
        rn = jnp.sqrt(jnp.sum(v * v, axis=1, keepdims=True))
        return v / (rn + 1e-12)

    sreg = region_vecs(sp_ref)
    treg = region_vecs(tp_ref)

    X = xq_ref[...]
    dn = (((1,), (1,)), ((), ()))
    t_c = jnp.float32(T_C)
    slog = lax.dot_general(sreg, X, dn,
                           preferred_element_type=jnp.float32) / t_c
    tlog = lax.dot_general(treg, X, dn,
                           preferred_element_type=jnp.float32) / t_c

    def logsoftmax(z):
        z = z - jnp.max(z, axis=1, keepdims=True)
        return z - jnp.log(jnp.sum(jnp.exp(z), axis=1, keepdims=True))

    log_ps = logsoftmax(slog)
    log_pt = logsoftmax(tlog)
    pt = jnp.exp(log_pt)
    kl = jnp.sum(pt * (log_pt - log_ps), axis=1, keepdims=True)

    p = jnp.clip(spd_ref[...], 1e-6, 1.0 - 1e-6)
    g = gt_ref[...]
    bce = -jnp.mean(g * jnp.log(p) + (1.0 - g) * jnp.log(1.0 - p),
                    axis=1, keepdims=True)
    inter = jnp.sum(p * g, axis=1, keepdims=True)
    dice = 1.0 - (2.0 * inter + 1e-5) / (
        jnp.sum(p, axis=1, keepdims=True)
        + jnp.sum(g, axis=1, keepdims=True) + 1e-5)
    st = 0.5 * bce + 0.5 * dice

    keep = vs_ref[...] > 0
    inf = jnp.float32(jnp.inf)
    xv = jnp.where(keep, st, inf)
    n_keep = jnp.sum(keep.astype(jnp.int32))
    q = jnp.clip((n_keep.astype(jnp.float32) * 0.6).astype(jnp.int32),
                 0, NREG - 1)
    eye = (lax.broadcasted_iota(jnp.int32, (NREG, NREG), 0)
           == lax.broadcasted_iota(jnp.int32, (NREG, NREG), 1))
    xt = jnp.sum(jnp.where(eye, xv, 0.0), axis=0, keepdims=True)
    cl = jnp.sum((xt < xv).astype(jnp.int32), axis=1, keepdims=True)
    ce = jnp.sum((xt == xv).astype(jnp.int32), axis=1, keepdims=True)
    okq = (cl <= q) & (q < cl + ce)
    min_c = jnp.min(jnp.where(okq, xv, inf))
    max_c = jnp.max(jnp.where(keep, st, -inf))
    msk = (keep & (st >= min_c) & (st <= max_c)).astype(jnp.float32)
    loss = jnp.sum(kl * msk) / jnp.maximum(jnp.sum(msk), 1.0)
    out_ref[...] = jnp.reshape(loss * jnp.float32(T_KD * T_KD), (1, 1))


def _finish_call(s_p, t_p, sp_p, gt_p, vs, xq, interpret=False):
    return pl.pallas_call(
        _finish_body,
        out_shape=jax.ShapeDtypeStruct((1, 1), jnp.float32),
        interpret=interpret,
    )(s_p, t_p, sp_p, gt_p, vs, xq)


def kernel(s_feats, t_feats, labels, teacher_predict, student_predict,
           queue, epoch, max_region_num):
    vs, fidx, pidx = _screen_call(labels)

    fidx3 = fidx.reshape(NREG, N_CHUNK, IDX_CHUNK)
    s_p, t_p, sp_p, gt_p = _sc_gather(
        fidx3, pidx,
        s_feats.reshape(-1), t_feats.reshape(-1),
        student_predict.reshape(-1), labels.reshape(-1))

    xq = queue[:, :CONTRAST_SIZE, :].reshape(NCON, DIM)
    res = _finish_call(s_p.reshape(NREG, DIM), t_p.reshape(NREG, DIM),
                       sp_p, gt_p, vs, xq)
    loss = res[0, 0]
    return jnp.where(epoch < WARMUP, jnp.float32(1e-9), loss)


# speedup vs baseline: 75.5967x; 1.5901x over previous
import functools

import jax
import jax.numpy as jnp
from jax import lax
from jax.experimental import pallas as pl
from jax.experimental.pallas import tpu as pltpu
from jax.experimental.pallas import tpu_sc as plsc

WIN = 8
K_PER = 16
NUM_CLASSES = 3
QUEUE_SIZE = 4096
CONTRAST_SIZE = 512
DIM = 32 * WIN * WIN
T_KD = 1.0
T_C = 0.1
WARMUP = 10

B = 4
H = 384
W = 384
NMAP = B * NUM_CLASSES
NREG = NMAP * K_PER
NCON = NUM_CLASSES * CONTRAST_SIZE
PLANE = H * W

SC_CORES = 2
SC_SUBCORES = 16
SC_WORKERS = SC_CORES * SC_SUBCORES
REG_PER_WORKER = NREG // SC_WORKERS
IDX_CHUNK = 128
N_CHUNK = DIM // IDX_CHUNK


def _screen_body(labels_ref, vs_ref, meta_ref):
    lesion = labels_ref[:, 1:1 + NUM_CLASSES].reshape(NMAP, H, W)
    nz = (lesion > 0.0).astype(jnp.float32)

    a = nz + jnp.roll(nz, -1, axis=2)
    a = a + jnp.roll(a, -2, axis=2)
    a = a + jnp.roll(a, -4, axis=2)
    cnt = a + jnp.roll(a, -1, axis=1)
    cnt = cnt + jnp.roll(cnt, -2, axis=1)
    cnt = cnt + jnp.roll(cnt, -4, axis=1)

    row_i = lax.broadcasted_iota(jnp.int32, (1, H, 1), 1)
    col_i = lax.broadcasted_iota(jnp.int32, (1, 1, W), 2)
    Hs = H - WIN + 1
    valid = (row_i < Hs) & (col_i < Hs)
    m_i = lax.broadcasted_iota(jnp.int32, (NMAP, 1, 1), 0)
    thr64 = jnp.where(m_i % NUM_CLASSES == 2,
                      jnp.float32(0.6) * jnp.float32(64.0),
                      jnp.float32(0.8) * jnp.float32(64.0))
    neg_inf = jnp.float32(-jnp.inf)

    C = jnp.float32(262144.0)
    flat_f = (row_i * W + col_i).astype(jnp.float32)
    kval = jnp.where(cnt > thr64, cnt, -1.0)
    enc = jnp.where(valid, kval * C - flat_f, neg_inf)

    lane16 = lax.broadcasted_iota(jnp.int32, (NMAP, K_PER), 1)
    row_f = row_i.astype(jnp.float32)
    col_f = col_i.astype(jnp.float32)

    def body(it, carry):
        enc, colmax, ys, xs, vsv = carry
        m = jnp.max(colmax, axis=2, keepdims=True)
        k = jnp.ceil(m * (1.0 / 262144.0))
        flat = k * C - m
        y = jnp.floor(flat / jnp.float32(W))
        x = flat - y * W
        v = jnp.where(k >= 0, k * (1.0 / (WIN * WIN)), -1.0)
        ov = (jnp.abs(row_f - y) < WIN) & (jnp.abs(col_f - x) < WIN)
        enc = jnp.where(ov, neg_inf, enc)
        colmax = jnp.max(enc, axis=1, keepdims=True)
        sel = lane16 == it
        ys = jnp.where(sel, y[:, :, 0], ys)
        xs = jnp.where(sel, x[:, :, 0], xs)
        vsv = jnp.where(sel, v[:, :, 0], vsv)
        return enc, colmax, ys, xs, vsv

    zf = jnp.zeros((NMAP, K_PER), jnp.float32)
    colmax0 = jnp.max(enc, axis=1, keepdims=True)
    _, _, ys, xs, vsv = lax.fori_loop(0, K_PER, body,
                                      (enc, colmax0, zf, zf, zf))

    oh_m = (lax.broadcasted_iota(jnp.int32, (NREG, NMAP), 1)
            == lax.broadcasted_iota(jnp.int32, (NREG, NMAP), 0) // K_PER
            ).astype(jnp.float32)
    oh_k = (lax.broadcasted_iota(jnp.int32, (NREG, K_PER), 1)
            == lax.broadcasted_iota(jnp.int32, (NREG, K_PER), 0) % K_PER
            ).astype(jnp.float32)
    dn2 = (((1,), (0,)), ((), ()))

    def flatten_sel(a):
        rows = lax.dot_general(oh_m, a, dn2,
                               preferred_element_type=jnp.float32)
        return jnp.sum(rows * oh_k, axis=1, keepdims=True)

    vr = flatten_sel(vsv)
    vs_ref[...] = vr
    keepr = vr > 0
    yr = jnp.where(keepr, flatten_sel(ys), 0.0).astype(jnp.int32)
    xr = jnp.where(keepr, flatten_sel(xs), 0.0).astype(jnp.int32)

    i3 = lax.broadcasted_iota(jnp.int32, (NREG, 16), 0)
    c3 = lax.broadcasted_iota(jnp.int32, (NREG, 16), 1)
    b3 = i3 // (NUM_CLASSES * K_PER)
    l3 = (i3 // K_PER) % NUM_CLASSES
    y0 = jnp.minimum((yr >> 3) << 3, H - 16)
    x0 = jnp.minimum((xr >> 7) << 7, 128)
    meta = jnp.where(c3 == 0, b3,
                     jnp.where(c3 == 1, y0,
                               jnp.where(c3 == 2, x0,
                                         jnp.where(c3 == 3, l3 + 1,
                                                   jnp.where(c3 == 4, yr - y0,
                                                             xr - x0)))))
    meta_ref[...] = jnp.where(c3 >= 6, 0, meta)


def _screen_call(labels, interpret=False):
    return pl.pallas_call(
        _screen_body,
        out_shape=[
            jax.ShapeDtypeStruct((NREG, 1), jnp.float32),
            jax.ShapeDtypeStruct((NREG, 16), jnp.int32),
        ],
        interpret=interpret,
    )(labels)


CCH = 4
NCHUNK_CH = 32 // CCH
VPC = CCH * WIN * WIN // 16


def _sc_gather_body(meta_hbm, s4_hbm, t4_hbm, pr4_hbm, lb4_hbm,
                    so_hbm, to_hbm, spo_hbm, gto_hbm,
                    mv, sbuf, tbuf, pbuf, gbuf, sout, tout, pout, gout,
                    sem_s, sem_t, sem_p):
    wid = lax.axis_index("s") * SC_CORES + lax.axis_index("c")
    lane = lax.iota(jnp.int32, 16)

    def do_region(j, _):
        i = wid * REG_PER_WORKER + j
        pltpu.sync_copy(meta_hbm.at[i], mv)
        mvec = mv[...]
        b = mvec[0]
        y0 = pl.multiple_of(mvec[1], 8)
        x0 = pl.multiple_of(mvec[2], 128)
        lp = mvec[3]
        dy = mvec[4]
        dx = mvec[5]

        def fire(t4, buf, sem, c0):
            return pltpu.async_copy(
                t4.at[b, pl.ds(c0 * CCH, CCH), pl.ds(y0, 16), pl.ds(x0, 256)],
                buf, sem)

        def extract(buf, out, c0):
            for v in range(VPC):
                e = c0 * (CCH * WIN * WIN) + v * 16 + lane
                c_loc = (e >> 6) - c0 * CCH
                r = (e >> 3) & 7
                k = e & 7
                vals = plsc.load_gather(buf, [c_loc, dy + r, dx + k])
                out[pl.ds(c0 * (CCH * WIN * WIN) + v * 16, 16)] = vals

        cp_s = fire(s4_hbm, sbuf, sem_s, 0)
        cp_t = fire(t4_hbm, tbuf, sem_t, 0)
        cp_p = pltpu.async_copy(
            pr4_hbm.at[b, lp, pl.ds(y0, 16), pl.ds(x0, 256)], pbuf, sem_p)
        cp_g = pltpu.async_copy(
            lb4_hbm.at[b, lp, pl.ds(y0, 16), pl.ds(x0, 256)], gbuf, sem_p)
        for c0 in range(NCHUNK_CH):
            cp_s.wait()
            extract(sbuf, sout, c0)
            if c0 + 1 < NCHUNK_CH:
                cp_s = fire(s4_hbm, sbuf, sem_s, c0 + 1)
            cp_t.wait()
            extract(tbuf, tout, c0)
            if c0 + 1 < NCHUNK_CH:
                cp_t = fire(t4_hbm, tbuf, sem_t, c0 + 1)
        cp_p.wait()
        cp_g.wait()
        for v in range(4):
            e = v * 16 + lane
            r = (e >> 3) & 7
            k = e & 7
            pout[pl.ds(v * 16, 16)] = plsc.load_gather(pbuf, [dy + r, dx + k])
            gout[pl.ds(v * 16, 16)] = plsc.load_gather(gbuf, [dy + r, dx + k])
        pltpu.sync_copy(sout, so_hbm.at[i])
        pltpu.sync_copy(tout, to_hbm.at[i])
        pltpu.sync_copy(pout, spo_hbm.at[i])
        pltpu.sync_copy(gout, gto_hbm.at[i])
        return 0

    lax.fori_loop(0, REG_PER_WORKER, do_region, 0)


def _sc_gather(meta, s_feats, t_feats, student_predict, labels):
    mesh = plsc.VectorSubcoreMesh(core_axis_name="c", subcore_axis_name="s")
    f = pl.kernel(
        _sc_gather_body,
        out_type=[
            jax.ShapeDtypeStruct((NREG, DIM), jnp.float32),
            jax.ShapeDtypeStruct((NREG, DIM), jnp.float32),
            jax.ShapeDtypeStruct((NREG, WIN * WIN), jnp.float32),
            jax.ShapeDtypeStruct((NREG, WIN * WIN), jnp.float32),
        ],
        mesh=mesh,
        compiler_params=pltpu.CompilerParams(needs_layout_passes=False),
        scratch_types=[
            pltpu.VMEM((16,), jnp.int32),
            pltpu.VMEM((CCH, 16, 256), jnp.float32),
            pltpu.VMEM((CCH, 16, 256), jnp.float32),
            pltpu.VMEM((16, 256), jnp.float32),
            pltpu.VMEM((16, 256), jnp.float32),
            pltpu.VMEM((DIM,), jnp.float32),
            pltpu.VMEM((DIM,), jnp.float32),
            pltpu.VMEM((WIN * WIN,), jnp.float32),
            pltpu.VMEM((WIN * WIN,), jnp.float32),
            pltpu.SemaphoreType.DMA,
            pltpu.SemaphoreType.DMA,
            pltpu.SemaphoreType.DMA,
        ],
    )
    return f(meta, s_feats, t_feats, student_predict, labels)


def _finish_body(sp_ref, tp_ref, spd_ref, gt_ref, vs_ref, xq_ref, out_ref):
    npix = WIN * WIN

    def region_vecs(ref):
        v = ref[...]
        v2 = v * v
        ss = v2[:, :npix]
        for c in range(1, 32):
            ss = ss + v2[:, c * npix:(c + 1) * npix]
        div = jnp.tile(jnp.sqrt(ss) + 1e-12, (1, 32))
        v = v / div
        rn = jnp.sqrt(jnp.sum(v * v, axis=1, keepdims=True))
        return v / (rn + 1e-12)

    sreg = region_vecs(sp_ref)
    treg = region_vecs(tp_ref)

    X = xq_ref[...]
    dn = (((1,), (1,)), ((), ()))
    t_c = jnp.float32(T_C)
    slog = lax.dot_general(sreg, X, dn,
                           preferred_element_type=jnp.float32) / t_c
    tlog = lax.dot_general(treg, X, dn,
                           preferred_element_type=jnp.float32) / t_c

    def logsoftmax(z):
        z = z - jnp.max(z, axis=1, keepdims=True)
        return z - jnp.log(jnp.sum(jnp.exp(z), axis=1, keepdims=True))

    log_ps = logsoftmax(slog)
    log_pt = logsoftmax(tlog)
    pt = jnp.exp(log_pt)
    kl = jnp.sum(pt * (log_pt - log_ps), axis=1, keepdims=True)

    p = jnp.clip(spd_ref[...], 1e-6, 1.0 - 1e-6)
    g = gt_ref[...]
    bce = -jnp.mean(g * jnp.log(p) + (1.0 - g) * jnp.log(1.0 - p),
                    axis=1, keepdims=True)
    inter = jnp.sum(p * g, axis=1, keepdims=True)
    dice = 1.0 - (2.0 * inter + 1e-5) / (
        jnp.sum(p, axis=1, keepdims=True)
        + jnp.sum(g, axis=1, keepdims=True) + 1e-5)
    st = 0.5 * bce + 0.5 * dice

    keep = vs_ref[...] > 0
    inf = jnp.float32(jnp.inf)
    xv = jnp.where(keep, st, inf)
    n_keep = jnp.sum(keep.astype(jnp.int32))
    q = jnp.clip((n_keep.astype(jnp.float32) * 0.6).astype(jnp.int32),
                 0, NREG - 1)
    eye = (lax.broadcasted_iota(jnp.int32, (NREG, NREG), 0)
           == lax.broadcasted_iota(jnp.int32, (NREG, NREG), 1))
    xt = jnp.sum(jnp.where(eye, xv, 0.0), axis=0, keepdims=True)
    cl = jnp.sum((xt < xv).astype(jnp.int32), axis=1, keepdims=True)
    ce = jnp.sum((xt == xv).astype(jnp.int32), axis=1, keepdims=True)
    okq = (cl <= q) & (q < cl + ce)
    min_c = jnp.min(jnp.where(okq, xv, inf))
    max_c = jnp.max(jnp.where(keep, st, -inf))
    msk = (keep & (st >= min_c) & (st <= max_c)).astype(jnp.float32)
    loss = jnp.sum(kl * msk) / jnp.maximum(jnp.sum(msk), 1.0)
    out_ref[...] = jnp.reshape(loss * jnp.float32(T_KD * T_KD), (1, 1))


def _finish_call(s_p, t_p, sp_p, gt_p, vs, xq, interpret=False):
    return pl.pallas_call(
        _finish_body,
        out_shape=jax.ShapeDtypeStruct((1, 1), jnp.float32),
        interpret=interpret,
    )(s_p, t_p, sp_p, gt_p, vs, xq)


def kernel(s_feats, t_feats, labels, teacher_predict, student_predict,
           queue, epoch, max_region_num):
    vs, meta = _screen_call(labels)

    s_p, t_p, sp_p, gt_p = _sc_gather(
        meta, s_feats, t_feats, student_predict, labels)

    xq = queue[:, :CONTRAST_SIZE, :].reshape(NCON, DIM)
    res = _finish_call(s_p, t_p, sp_p, gt_p, vs, xq)
    loss = res[0, 0]
    return jnp.where(epoch < WARMUP, jnp.float32(1e-9), loss)


# speedup vs baseline: 94.2801x; 1.2471x over previous
import functools

import jax
import jax.numpy as jnp
from jax import lax
from jax.experimental import pallas as pl
from jax.experimental.pallas import tpu as pltpu
from jax.experimental.pallas import tpu_sc as plsc

WIN = 8
K_PER = 16
NUM_CLASSES = 3
QUEUE_SIZE = 4096
CONTRAST_SIZE = 512
DIM = 32 * WIN * WIN
T_KD = 1.0
T_C = 0.1
WARMUP = 10

B = 4
H = 384
W = 384
NMAP = B * NUM_CLASSES
NREG = NMAP * K_PER
NCON = NUM_CLASSES * CONTRAST_SIZE
PLANE = H * W

SC_CORES = 2
SC_SUBCORES = 16
SC_WORKERS = SC_CORES * SC_SUBCORES
REG_PER_WORKER = NREG // SC_WORKERS
IDX_CHUNK = 128
N_CHUNK = DIM // IDX_CHUNK


def _screen_body(labels_ref, vs_ref, meta_ref):
    lesion = labels_ref[:, 1:1 + NUM_CLASSES].reshape(NMAP, H, W)
    nz = (lesion > 0.0).astype(jnp.float32)

    a = nz + jnp.roll(nz, -1, axis=2)
    a = a + jnp.roll(a, -2, axis=2)
    a = a + jnp.roll(a, -4, axis=2)
    cnt = a + jnp.roll(a, -1, axis=1)
    cnt = cnt + jnp.roll(cnt, -2, axis=1)
    cnt = cnt + jnp.roll(cnt, -4, axis=1)

    row_i = lax.broadcasted_iota(jnp.int32, (1, H, 1), 1)
    col_i = lax.broadcasted_iota(jnp.int32, (1, 1, W), 2)
    Hs = H - WIN + 1
    valid = (row_i < Hs) & (col_i < Hs)
    m_i = lax.broadcasted_iota(jnp.int32, (NMAP, 1, 1), 0)
    thr64 = jnp.where(m_i % NUM_CLASSES == 2,
                      jnp.float32(0.6) * jnp.float32(64.0),
                      jnp.float32(0.8) * jnp.float32(64.0))
    neg_inf = jnp.float32(-jnp.inf)

    C = jnp.float32(262144.0)
    flat_f = (row_i * W + col_i).astype(jnp.float32)
    kval = jnp.where(cnt > thr64, cnt, -1.0)
    enc = jnp.where(valid, kval * C - flat_f, neg_inf)

    lane16 = lax.broadcasted_iota(jnp.int32, (NMAP, K_PER), 1)
    row_f = row_i.astype(jnp.float32)
    col_f = col_i.astype(jnp.float32)

    def body(it, carry):
        enc, colmax, ys, xs, vsv = carry
        m = jnp.max(colmax, axis=2, keepdims=True)
        k = jnp.ceil(m * (1.0 / 262144.0))
        flat = k * C - m
        y = jnp.floor(flat / jnp.float32(W))
        x = flat - y * W
        v = jnp.where(k >= 0, k * (1.0 / (WIN * WIN)), -1.0)
        ov = (jnp.abs(row_f - y) < WIN) & (jnp.abs(col_f - x) < WIN)
        enc = jnp.where(ov, neg_inf, enc)
        colmax = jnp.max(enc, axis=1, keepdims=True)
        sel = lane16 == it
        ys = jnp.where(sel, y[:, :, 0], ys)
        xs = jnp.where(sel, x[:, :, 0], xs)
        vsv = jnp.where(sel, v[:, :, 0], vsv)
        return enc, colmax, ys, xs, vsv

    zf = jnp.zeros((NMAP, K_PER), jnp.float32)
    colmax0 = jnp.max(enc, axis=1, keepdims=True)
    _, _, ys, xs, vsv = lax.fori_loop(0, K_PER, body,
                                      (enc, colmax0, zf, zf, zf))

    oh_m = (lax.broadcasted_iota(jnp.int32, (NREG, NMAP), 1)
            == lax.broadcasted_iota(jnp.int32, (NREG, NMAP), 0) // K_PER
            ).astype(jnp.float32)
    oh_k = (lax.broadcasted_iota(jnp.int32, (NREG, K_PER), 1)
            == lax.broadcasted_iota(jnp.int32, (NREG, K_PER), 0) % K_PER
            ).astype(jnp.float32)
    dn2 = (((1,), (0,)), ((), ()))

    def flatten_sel(a):
        rows = lax.dot_general(oh_m, a, dn2,
                               preferred_element_type=jnp.float32)
        return jnp.sum(rows * oh_k, axis=1, keepdims=True)

    vr = flatten_sel(vsv)
    vs_ref[...] = vr
    keepr = vr > 0
    yr = jnp.where(keepr, flatten_sel(ys), 0.0).astype(jnp.int32)
    xr = jnp.where(keepr, flatten_sel(xs), 0.0).astype(jnp.int32)

    i3 = lax.broadcasted_iota(jnp.int32, (NREG, 16), 0)
    c3 = lax.broadcasted_iota(jnp.int32, (NREG, 16), 1)
    b3 = i3 // (NUM_CLASSES * K_PER)
    l3 = (i3 // K_PER) % NUM_CLASSES
    y0 = jnp.minimum((yr >> 3) << 3, H - 16)
    x0 = (xr >> 7) << 7
    dx = xr - x0
    meta = jnp.where(c3 == 0, b3,
                     jnp.where(c3 == 1, y0,
                               jnp.where(c3 == 2, x0,
                                         jnp.where(c3 == 3, l3 + 1,
                                                   jnp.where(c3 == 4, yr - y0,
                                                             jnp.where(c3 == 5, dx,
                                                                       (dx > 120).astype(jnp.int32)))))))
    meta_ref[...] = jnp.where(c3 >= 7, 0, meta)


def _screen_call(labels, interpret=False):
    return pl.pallas_call(
        _screen_body,
        out_shape=[
            jax.ShapeDtypeStruct((NREG, 1), jnp.float32),
            jax.ShapeDtypeStruct((NREG, 16), jnp.int32),
        ],
        interpret=interpret,
    )(labels)


CCH = 8
NCHUNK_CH = 32 // CCH
VPC = CCH * WIN * WIN // 16


def _sc_gather_body(meta_hbm, s4_hbm, t4_hbm, pr4_hbm, lb4_hbm,
                    so_hbm, to_hbm, spo_hbm, gto_hbm,
                    mv, sbuf, tbuf, pbuf, gbuf, sout, tout, pout, gout,
                    sem_s, sem_t, sem_p):
    wid = lax.axis_index("s") * SC_CORES + lax.axis_index("c")
    lane = lax.iota(jnp.int32, 16)

    def do_region(j, _):
        i = wid * REG_PER_WORKER + j
        pltpu.sync_copy(meta_hbm.at[i], mv)
        mvec = mv[...]
        b = mvec[0]
        y0 = pl.multiple_of(mvec[1], 8)
        x0 = pl.multiple_of(mvec[2], 128)
        x1 = pl.multiple_of(x0 + 128, 128)
        lp = mvec[3]
        dy = mvec[4]
        dx = mvec[5]
        two = mvec[6] > 0

        def fire(t4, buf, sem, c0):
            cs = pl.ds(c0 * CCH, CCH)
            cpa = pltpu.async_copy(
                t4.at[b, cs, pl.ds(y0, 16), pl.ds(x0, 128)], buf.at[0], sem)

            @pl.when(two)
            def _():
                pltpu.async_copy(
                    t4.at[b, cs, pl.ds(y0, 16), pl.ds(x1, 128)],
                    buf.at[1], sem)
            return cpa

        def drain(cpa, t4, buf, sem, c0):
            cpa.wait()

            @pl.when(two)
            def _():
                pltpu.make_async_copy(
                    t4.at[b, pl.ds(0, CCH), pl.ds(y0, 16), pl.ds(x1, 128)],
                    buf.at[1], sem).wait()

        def extract(buf, out, c0):
            for v in range(VPC):
                e = c0 * (CCH * WIN * WIN) + v * 16 + lane
                c_loc = (e >> 6) - c0 * CCH
                r = (e >> 3) & 7
                cx = dx + (e & 7)
                vals = plsc.load_gather(
                    buf, [cx >> 7, c_loc, dy + r, cx & 127])
                out[pl.ds(c0 * (CCH * WIN * WIN) + v * 16, 16)] = vals

        cp_s = fire(s4_hbm, sbuf, sem_s, 0)
        cp_t = fire(t4_hbm, tbuf, sem_t, 0)
        cp_p = pltpu.async_copy(
            pr4_hbm.at[b, lp, pl.ds(y0, 16), pl.ds(x0, 128)], pbuf.at[0],
            sem_p)
        cp_g = pltpu.async_copy(
            lb4_hbm.at[b, lp, pl.ds(y0, 16), pl.ds(x0, 128)], gbuf.at[0],
            sem_p)

        @pl.when(two)
        def _():
            pltpu.async_copy(
                pr4_hbm.at[b, lp, pl.ds(y0, 16), pl.ds(x1, 128)],
                pbuf.at[1], sem_p)
            pltpu.async_copy(
                lb4_hbm.at[b, lp, pl.ds(y0, 16), pl.ds(x1, 128)],
                gbuf.at[1], sem_p)

        for c0 in range(NCHUNK_CH):
            drain(cp_s, s4_hbm, sbuf, sem_s, c0)
            extract(sbuf, sout, c0)
            if c0 + 1 < NCHUNK_CH:
                cp_s = fire(s4_hbm, sbuf, sem_s, c0 + 1)
            drain(cp_t, t4_hbm, tbuf, sem_t, c0)
            extract(tbuf, tout, c0)
            if c0 + 1 < NCHUNK_CH:
                cp_t = fire(t4_hbm, tbuf, sem_t, c0 + 1)

        cp_p.wait()
        cp_g.wait()

        @pl.when(two)
        def _():
            pltpu.make_async_copy(
                pr4_hbm.at[b, lp, pl.ds(y0, 16), pl.ds(x1, 128)],
                pbuf.at[1], sem_p).wait()
            pltpu.make_async_copy(
                lb4_hbm.at[b, lp, pl.ds(y0, 16), pl.ds(x1, 128)],
                gbuf.at[1], sem_p).wait()

        for v in range(4):
            e = v * 16 + lane
            r = (e >> 3) & 7
            cx = dx + (e & 7)
            pout[pl.ds(v * 16, 16)] = plsc.load_gather(
                pbuf, [cx >> 7, dy + r, cx & 127])
            gout[pl.ds(v * 16, 16)] = plsc.load_gather(
                gbuf, [cx >> 7, dy + r, cx & 127])
        pltpu.sync_copy(sout, so_hbm.at[i])
        pltpu.sync_copy(tout, to_hbm.at[i])
        pltpu.sync_copy(pout, spo_hbm.at[i])
        pltpu.sync_copy(gout, gto_hbm.at[i])
        return 0

    lax.fori_loop(0, REG_PER_WORKER, do_region, 0)


def _sc_gather(meta, s_feats, t_feats, student_predict, labels):
    mesh = plsc.VectorSubcoreMesh(core_axis_name="c", subcore_axis_name="s")
    f = pl.kernel(
        _sc_gather_body,
        out_type=[
            jax.ShapeDtypeStruct((NREG, DIM), jnp.float32),
            jax.ShapeDtypeStruct((NREG, DIM), jnp.float32),
            jax.ShapeDtypeStruct((NREG, WIN * WIN), jnp.float32),
            jax.ShapeDtypeStruct((NREG, WIN * WIN), jnp.float32),
        ],
        mesh=mesh,
        compiler_params=pltpu.CompilerParams(needs_layout_passes=False),
        scratch_types=[
            pltpu.VMEM((16,), jnp.int32),
            pltpu.VMEM((2, CCH, 16, 128), jnp.float32),
            pltpu.VMEM((2, CCH, 16, 128), jnp.float32),
            pltpu.VMEM((2, 16, 128), jnp.float32),
            pltpu.VMEM((2, 16, 128), jnp.float32),
            pltpu.VMEM((DIM,), jnp.float32),
            pltpu.VMEM((DIM,), jnp.float32),
            pltpu.VMEM((WIN * WIN,), jnp.float32),
            pltpu.VMEM((WIN * WIN,), jnp.float32),
            pltpu.SemaphoreType.DMA,
            pltpu.SemaphoreType.DMA,
            pltpu.SemaphoreType.DMA,
        ],
    )
    return f(meta, s_feats, t_feats, student_predict, labels)


def _finish_body(sp_ref, tp_ref, spd_ref, gt_ref, vs_ref, xq_ref, out_ref):
    npix = WIN * WIN

    def region_vecs(ref):
        v = ref[...]
        v2 = v * v
        ss = v2[:, :npix]
        for c in range(1, 32):
            ss = ss + v2[:, c * npix:(c + 1) * npix]
        div = jnp.tile(jnp.sqrt(ss) + 1e-12, (1, 32))
        v = v / div
        rn = jnp.sqrt(jnp.sum(v * v, axis=1, keepdims=True))
        return v / (rn + 1e-12)

    sreg = region_vecs(sp_ref)
    treg = region_vecs(tp_ref)

    X = xq_ref[...]
    dn = (((1,), (1,)), ((), ()))
    t_c = jnp.float32(T_C)
    slog = lax.dot_general(sreg, X, dn,
                           preferred_element_type=jnp.float32) / t_c
    tlog = lax.dot_general(treg, X, dn,
                           preferred_element_type=jnp.float32) / t_c

    def logsoftmax(z):
        z = z - jnp.max(z, axis=1, keepdims=True)
        return z - jnp.log(jnp.sum(jnp.exp(z), axis=1, keepdims=True))

    log_ps = logsoftmax(slog)
    log_pt = logsoftmax(tlog)
    pt = jnp.exp(log_pt)
    kl = jnp.sum(pt * (log_pt - log_ps), axis=1, keepdims=True)

    p = jnp.clip(spd_ref[...], 1e-6, 1.0 - 1e-6)
    g = gt_ref[...]
    bce = -jnp.mean(g * jnp.log(p) + (1.0 - g) * jnp.log(1.0 - p),
                    axis=1, keepdims=True)
    inter = jnp.sum(p * g, axis=1, keepdims=True)
    dice = 1.0 - (2.0 * inter + 1e-5) / (
        jnp.sum(p, axis=1, keepdims=True)
        + jnp.sum(g, axis=1, keepdims=True) + 1e-5)
    st = 0.5 * bce + 0.5 * dice

    keep = vs_ref[...] > 0
    inf = jnp.float32(jnp.inf)
    xv = jnp.where(keep, st, inf)
    n_keep = jnp.sum(keep.astype(jnp.int32))
    q = jnp.clip((n_keep.astype(jnp.float32) * 0.6).astype(jnp.int32),
                 0, NREG - 1)
    eye = (lax.broadcasted_iota(jnp.int32, (NREG, NREG), 0)
           == lax.broadcasted_iota(jnp.int32, (NREG, NREG), 1))
    xt = jnp.sum(jnp.where(eye, xv, 0.0), axis=0, keepdims=True)
    cl = jnp.sum((xt < xv).astype(jnp.int32), axis=1, keepdims=True)
    ce = jnp.sum((xt == xv).astype(jnp.int32), axis=1, keepdims=True)
    okq = (cl <= q) & (q < cl + ce)
    min_c = jnp.min(jnp.where(okq, xv, inf))
    max_c = jnp.max(jnp.where(keep, st, -inf))
    msk = (keep & (st >= min_c) & (st <= max_c)).astype(jnp.float32)
    loss = jnp.sum(kl * msk) / jnp.maximum(jnp.sum(msk), 1.0)
    out_ref[...] = jnp.reshape(loss * jnp.float32(T_KD * T_KD), (1, 1))


def _finish_call(s_p, t_p, sp_p, gt_p, vs, xq, interpret=False):
    return pl.pallas_call(
        _finish_body,
        out_shape=jax.ShapeDtypeStruct((1, 1), jnp.float32),
        interpret=interpret,
    )(s_p, t_p, sp_p, gt_p, vs, xq)


def kernel(s_feats, t_feats, labels, teacher_predict, student_predict,
           queue, epoch, max_region_num):
    vs, meta = _screen_call(labels)

    s_p, t_p, sp_p, gt_p = _sc_gather(
        meta, s_feats, t_feats, student_predict, labels)

    xq = queue[:, :CONTRAST_SIZE, :].reshape(NCON, DIM)
    res = _finish_call(s_p, t_p, sp_p, gt_p, vs, xq)
    loss = res[0, 0]
    return jnp.where(epoch < WARMUP, jnp.float32(1e-9), loss)


# speedup vs baseline: 105.0310x; 1.1140x over previous
import functools

import jax
import jax.numpy as jnp
from jax import lax
from jax.experimental import pallas as pl
from jax.experimental.pallas import tpu as pltpu
from jax.experimental.pallas import tpu_sc as plsc

WIN = 8
K_PER = 16
NUM_CLASSES = 3
QUEUE_SIZE = 4096
CONTRAST_SIZE = 512
DIM = 32 * WIN * WIN
T_KD = 1.0
T_C = 0.1
WARMUP = 10

B = 4
H = 384
W = 384
NMAP = B * NUM_CLASSES
NREG = NMAP * K_PER
NCON = NUM_CLASSES * CONTRAST_SIZE
PLANE = H * W

SC_CORES = 2
SC_SUBCORES = 16
SC_WORKERS = SC_CORES * SC_SUBCORES
REG_PER_WORKER = NREG // 2 // SC_WORKERS
IDX_CHUNK = 128
N_CHUNK = DIM // IDX_CHUNK


NMAP_H = NMAP // 2
NREG_H = NREG // 2


def _screen_body(half, labels_ref, vs_ref, meta_ref):
    lesion = labels_ref[:, 1:1 + NUM_CLASSES].reshape(NMAP_H, H, W)
    nz = (lesion > 0.0).astype(jnp.float32)

    a = nz + jnp.roll(nz, -1, axis=2)
    a = a + jnp.roll(a, -2, axis=2)
    a = a + jnp.roll(a, -4, axis=2)
    cnt = a + jnp.roll(a, -1, axis=1)
    cnt = cnt + jnp.roll(cnt, -2, axis=1)
    cnt = cnt + jnp.roll(cnt, -4, axis=1)

    row_i = lax.broadcasted_iota(jnp.int32, (1, H, 1), 1)
    col_i = lax.broadcasted_iota(jnp.int32, (1, 1, W), 2)
    Hs = H - WIN + 1
    valid = (row_i < Hs) & (col_i < Hs)
    m_i = lax.broadcasted_iota(jnp.int32, (NMAP_H, 1, 1), 0)
    thr64 = jnp.where(m_i % NUM_CLASSES == 2,
                      jnp.float32(0.6) * jnp.float32(64.0),
                      jnp.float32(0.8) * jnp.float32(64.0))
    neg_inf = jnp.float32(-jnp.inf)

    C = jnp.float32(262144.0)
    flat_f = (row_i * W + col_i).astype(jnp.float32)
    kval = jnp.where(cnt > thr64, cnt, -1.0)
    enc = jnp.where(valid, kval * C - flat_f, neg_inf)

    lane16 = lax.broadcasted_iota(jnp.int32, (NMAP_H, K_PER), 1)
    row_f = row_i.astype(jnp.float32)
    col_f = col_i.astype(jnp.float32)

    def body(it, carry):
        enc, colmax, ys, xs, vsv = carry
        m = jnp.max(colmax, axis=2, keepdims=True)
        k = jnp.ceil(m * (1.0 / 262144.0))
        flat = k * C - m
        y = jnp.floor(flat / jnp.float32(W))
        x = flat - y * W
        v = jnp.where(k >= 0, k * (1.0 / (WIN * WIN)), -1.0)
        ov = (jnp.abs(row_f - y) < WIN) & (jnp.abs(col_f - x) < WIN)
        enc = jnp.where(ov, neg_inf, enc)
        colmax = jnp.max(enc, axis=1, keepdims=True)
        sel = lane16 == it
        ys = jnp.where(sel, y[:, :, 0], ys)
        xs = jnp.where(sel, x[:, :, 0], xs)
        vsv = jnp.where(sel, v[:, :, 0], vsv)
        return enc, colmax, ys, xs, vsv

    zf = jnp.zeros((NMAP_H, K_PER), jnp.float32)
    colmax0 = jnp.max(enc, axis=1, keepdims=True)
    _, _, ys, xs, vsv = lax.fori_loop(0, K_PER, body,
                                      (enc, colmax0, zf, zf, zf))

    oh_m = (lax.broadcasted_iota(jnp.int32, (NREG_H, NMAP_H), 1)
            == lax.broadcasted_iota(jnp.int32, (NREG_H, NMAP_H), 0) // K_PER
            ).astype(jnp.float32)
    oh_k = (lax.broadcasted_iota(jnp.int32, (NREG_H, K_PER), 1)
            == lax.broadcasted_iota(jnp.int32, (NREG_H, K_PER), 0) % K_PER
            ).astype(jnp.float32)
    dn2 = (((1,), (0,)), ((), ()))

    def flatten_sel(a):
        rows = lax.dot_general(oh_m, a, dn2,
                               preferred_element_type=jnp.float32)
        return jnp.sum(rows * oh_k, axis=1, keepdims=True)

    vr = flatten_sel(vsv)
    vs_ref[...] = vr
    keepr = vr > 0
    yr = jnp.where(keepr, flatten_sel(ys), 0.0).astype(jnp.int32)
    xr = jnp.where(keepr, flatten_sel(xs), 0.0).astype(jnp.int32)

    i3 = lax.broadcasted_iota(jnp.int32, (NREG_H, 16), 0)
    c3 = lax.broadcasted_iota(jnp.int32, (NREG_H, 16), 1)
    b3 = i3 // (NUM_CLASSES * K_PER) + 2 * half
    l3 = (i3 // K_PER) % NUM_CLASSES
    y0 = jnp.minimum((yr >> 3) << 3, H - 16)
    x0 = (xr >> 7) << 7
    dx = xr - x0
    meta = jnp.where(c3 == 0, b3,
                     jnp.where(c3 == 1, y0,
                               jnp.where(c3 == 2, x0,
                                         jnp.where(c3 == 3, l3 + 1,
                                                   jnp.where(c3 == 4, yr - y0,
                                                             jnp.where(c3 == 5, dx,
                                                                       (dx > 120).astype(jnp.int32)))))))
    meta_ref[...] = jnp.where(c3 >= 7, 0, meta)


def _screen_call(labels, half, interpret=False):
    return pl.pallas_call(
        functools.partial(_screen_body, half),
        grid=(1,),
        in_specs=[pl.BlockSpec((2, 5, H, W), lambda i, h=half: (h, 0, 0, 0))],
        out_specs=[
            pl.BlockSpec((NREG_H, 1), lambda i: (0, 0)),
            pl.BlockSpec((NREG_H, 16), lambda i: (0, 0)),
        ],
        out_shape=[
            jax.ShapeDtypeStruct((NREG_H, 1), jnp.float32),
            jax.ShapeDtypeStruct((NREG_H, 16), jnp.int32),
        ],
        interpret=interpret,
    )(labels)


CCH = 8
NCHUNK_CH = 32 // CCH
VPC = CCH * WIN * WIN // 16


def _sc_gather_body(meta_hbm, s4_hbm, t4_hbm, pr4_hbm, lb4_hbm,
                    so_hbm, to_hbm, spo_hbm, gto_hbm,
                    mv, sbuf, tbuf, pbuf, gbuf, sout, tout, pout, gout,
                    sem_s, sem_t, sem_p):
    wid = lax.axis_index("s") * SC_CORES + lax.axis_index("c")
    lane = lax.iota(jnp.int32, 16)

    def do_region(j, _):
        i = wid * REG_PER_WORKER + j
        pltpu.sync_copy(meta_hbm.at[i], mv)
        mvec = mv[...]
        b = mvec[0]
        y0 = pl.multiple_of(mvec[1], 8)
        x0 = pl.multiple_of(mvec[2], 128)
        x1 = pl.multiple_of(x0 + 128, 128)
        lp = mvec[3]
        dy = mvec[4]
        dx = mvec[5]
        two = mvec[6] > 0

        def fire(t4, buf, sem, c0):
            cs = pl.ds(c0 * CCH, CCH)
            cpa = pltpu.async_copy(
                t4.at[b, cs, pl.ds(y0, 16), pl.ds(x0, 128)], buf.at[0], sem)

            @pl.when(two)
            def _():
                pltpu.async_copy(
                    t4.at[b, cs, pl.ds(y0, 16), pl.ds(x1, 128)],
                    buf.at[1], sem)
            return cpa

        def drain(cpa, t4, buf, sem, c0):
            cpa.wait()

            @pl.when(two)
            def _():
                pltpu.make_async_copy(
                    t4.at[b, pl.ds(0, CCH), pl.ds(y0, 16), pl.ds(x1, 128)],
                    buf.at[1], sem).wait()

        def extract(buf, out, c0):
            for v in range(VPC):
                e = c0 * (CCH * WIN * WIN) + v * 16 + lane
                c_loc = (e >> 6) - c0 * CCH
                r = (e >> 3) & 7
                cx = dx + (e & 7)
                vals = plsc.load_gather(
                    buf, [cx >> 7, c_loc, dy + r, cx & 127])
                out[pl.ds(c0 * (CCH * WIN * WIN) + v * 16, 16)] = vals

        cp_s = fire(s4_hbm, sbuf, sem_s, 0)
        cp_t = fire(t4_hbm, tbuf, sem_t, 0)
        cp_p = pltpu.async_copy(
            pr4_hbm.at[b, lp, pl.ds(y0, 16), pl.ds(x0, 128)], pbuf.at[0],
            sem_p)
        cp_g = pltpu.async_copy(
            lb4_hbm.at[b, lp, pl.ds(y0, 16), pl.ds(x0, 128)], gbuf.at[0],
            sem_p)

        @pl.when(two)
        def _():
            pltpu.async_copy(
                pr4_hbm.at[b, lp, pl.ds(y0, 16), pl.ds(x1, 128)],
                pbuf.at[1], sem_p)
            pltpu.async_copy(
                lb4_hbm.at[b, lp, pl.ds(y0, 16), pl.ds(x1, 128)],
                gbuf.at[1], sem_p)

        for c0 in range(NCHUNK_CH):
            drain(cp_s, s4_hbm, sbuf, sem_s, c0)
            extract(sbuf, sout, c0)
            if c0 + 1 < NCHUNK_CH:
                cp_s = fire(s4_hbm, sbuf, sem_s, c0 + 1)
            drain(cp_t, t4_hbm, tbuf, sem_t, c0)
            extract(tbuf, tout, c0)
            if c0 + 1 < NCHUNK_CH:
                cp_t = fire(t4_hbm, tbuf, sem_t, c0 + 1)

        cp_p.wait()
        cp_g.wait()

        @pl.when(two)
        def _():
            pltpu.make_async_copy(
                pr4_hbm.at[b, lp, pl.ds(y0, 16), pl.ds(x1, 128)],
                pbuf.at[1], sem_p).wait()
            pltpu.make_async_copy(
                lb4_hbm.at[b, lp, pl.ds(y0, 16), pl.ds(x1, 128)],
                gbuf.at[1], sem_p).wait()

        for v in range(4):
            e = v * 16 + lane
            r = (e >> 3) & 7
            cx = dx + (e & 7)
            pout[pl.ds(v * 16, 16)] = plsc.load_gather(
                pbuf, [cx >> 7, dy + r, cx & 127])
            gout[pl.ds(v * 16, 16)] = plsc.load_gather(
                gbuf, [cx >> 7, dy + r, cx & 127])
        pltpu.sync_copy(sout, so_hbm.at[i])
        pltpu.sync_copy(tout, to_hbm.at[i])
        pltpu.sync_copy(pout, spo_hbm.at[i])
        pltpu.sync_copy(gout, gto_hbm.at[i])
        return 0

    lax.fori_loop(0, REG_PER_WORKER, do_region, 0)


def _sc_gather(meta, s_feats, t_feats, student_predict, labels):
    mesh = plsc.VectorSubcoreMesh(core_axis_name="c", subcore_axis_name="s")
    f = pl.kernel(
        _sc_gather_body,
        out_type=[
            jax.ShapeDtypeStruct((NREG_H, DIM), jnp.float32),
            jax.ShapeDtypeStruct((NREG_H, DIM), jnp.float32),
            jax.ShapeDtypeStruct((NREG_H, WIN * WIN), jnp.float32),
            jax.ShapeDtypeStruct((NREG_H, WIN * WIN), jnp.float32),
        ],
        mesh=mesh,
        compiler_params=pltpu.CompilerParams(needs_layout_passes=False),
        scratch_types=[
            pltpu.VMEM((16,), jnp.int32),
            pltpu.VMEM((2, CCH, 16, 128), jnp.float32),
            pltpu.VMEM((2, CCH, 16, 128), jnp.float32),
            pltpu.VMEM((2, 16, 128), jnp.float32),
            pltpu.VMEM((2, 16, 128), jnp.float32),
            pltpu.VMEM((DIM,), jnp.float32),
            pltpu.VMEM((DIM,), jnp.float32),
            pltpu.VMEM((WIN * WIN,), jnp.float32),
            pltpu.VMEM((WIN * WIN,), jnp.float32),
            pltpu.SemaphoreType.DMA,
            pltpu.SemaphoreType.DMA,
            pltpu.SemaphoreType.DMA,
        ],
    )
    return f(meta, s_feats, t_feats, student_predict, labels)


def _finish_body(sp_ref, tp_ref, spd_ref, gt_ref, vs_ref, xq_ref, out_ref):
    npix = WIN * WIN

    def region_vecs(ref):
        v = ref[...]
        v2 = v * v
        ss = v2[:, :npix]
        for c in range(1, 32):
            ss = ss + v2[:, c * npix:(c + 1) * npix]
        div = jnp.tile(jnp.sqrt(ss) + 1e-12, (1, 32))
        v = v / div
        rn = jnp.sqrt(jnp.sum(v * v, axis=1, keepdims=True))
        return v / (rn + 1e-12)

    sreg = region_vecs(sp_ref)
    treg = region_vecs(tp_ref)

    X = xq_ref[...]
    dn = (((1,), (1,)), ((), ()))
    t_c = jnp.float32(T_C)
    slog = lax.dot_general(sreg, X, dn,
                           preferred_element_type=jnp.float32) / t_c
    tlog = lax.dot_general(treg, X, dn,
                           preferred_element_type=jnp.float32) / t_c

    def logsoftmax(z):
        z = z - jnp.max(z, axis=1, keepdims=True)
        return z - jnp.log(jnp.sum(jnp.exp(z), axis=1, keepdims=True))

    log_ps = logsoftmax(slog)
    log_pt = logsoftmax(tlog)
    pt = jnp.exp(log_pt)
    kl = jnp.sum(pt * (log_pt - log_ps), axis=1, keepdims=True)

    p = jnp.clip(spd_ref[...], 1e-6, 1.0 - 1e-6)
    g = gt_ref[...]
    bce = -jnp.mean(g * jnp.log(p) + (1.0 - g) * jnp.log(1.0 - p),
                    axis=1, keepdims=True)
    inter = jnp.sum(p * g, axis=1, keepdims=True)
    dice = 1.0 - (2.0 * inter + 1e-5) / (
        jnp.sum(p, axis=1, keepdims=True)
        + jnp.sum(g, axis=1, keepdims=True) + 1e-5)
    st = 0.5 * bce + 0.5 * dice

    keep = vs_ref[...] > 0
    inf = jnp.float32(jnp.inf)
    xv = jnp.where(keep, st, inf)
    n_keep = jnp.sum(keep.astype(jnp.int32))
    q = jnp.clip((n_keep.astype(jnp.float32) * 0.6).astype(jnp.int32),
                 0, NREG - 1)
    eye = (lax.broadcasted_iota(jnp.int32, (NREG, NREG), 0)
           == lax.broadcasted_iota(jnp.int32, (NREG, NREG), 1))
    xt = jnp.sum(jnp.where(eye, xv, 0.0), axis=0, keepdims=True)
    cl = jnp.sum((xt < xv).astype(jnp.int32), axis=1, keepdims=True)
    ce = jnp.sum((xt == xv).astype(jnp.int32), axis=1, keepdims=True)
    okq = (cl <= q) & (q < cl + ce)
    min_c = jnp.min(jnp.where(okq, xv, inf))
    max_c = jnp.max(jnp.where(keep, st, -inf))
    msk = (keep & (st >= min_c) & (st <= max_c)).astype(jnp.float32)
    loss = jnp.sum(kl * msk) / jnp.maximum(jnp.sum(msk), 1.0)
    out_ref[...] = jnp.reshape(loss * jnp.float32(T_KD * T_KD), (1, 1))


def _finish_call(s_p, t_p, sp_p, gt_p, vs, xq, interpret=False):
    return pl.pallas_call(
        _finish_body,
        out_shape=jax.ShapeDtypeStruct((1, 1), jnp.float32),
        interpret=interpret,
    )(s_p, t_p, sp_p, gt_p, vs, xq)


def kernel(s_feats, t_feats, labels, teacher_predict, student_predict,
           queue, epoch, max_region_num):
    vs_a, meta_a = _screen_call(labels, 0)
    ga = _sc_gather(meta_a, s_feats, t_feats, student_predict, labels)
    vs_b, meta_b = _screen_call(labels, 1)
    gb = _sc_gather(meta_b, s_feats, t_feats, student_predict, labels)

    s_p, t_p, sp_p, gt_p = (jnp.concatenate([a, b_], axis=0)
                            for a, b_ in zip(ga, gb))
    vs = jnp.concatenate([vs_a, vs_b], axis=0)
    xq = queue[:, :CONTRAST_SIZE, :].reshape(NCON, DIM)
    res = _finish_call(s_p, t_p, sp_p, gt_p, vs, xq)
    loss = res[0, 0]
    return jnp.where(epoch < WARMUP, jnp.float32(1e-9), loss)
